# baseline (speedup 1.0000x reference)
import jax
import jax.numpy as jnp
from jax import lax
from jax.experimental import pallas as pl
from jax.experimental.pallas import tpu as pltpu
from jax.experimental.pallas import tpu_sc as plsc

VOCAB = 1000
NC, NS = 2, 16
NW = NC * NS
CH = 32
LANES = 16


def _sc_onehot(ids_hbm, zeros_hbm, out_hbm, ids_v, buf0, buf1, sem0, sem1):
    n_rows = ids_hbm.shape[0]
    rows_per_w = n_rows // NW
    n_chunks = rows_per_w // CH
    wid = lax.axis_index("s") * NC + lax.axis_index("c")
    base = wid * rows_per_w

    pltpu.sync_copy(ids_hbm.at[pl.ds(base, rows_per_w)], ids_v)
    pltpu.sync_copy(zeros_hbm, buf0)
    pltpu.sync_copy(zeros_hbm, buf1)

    iota = lax.iota(jnp.int32, LANES)
    ones_v = jnp.ones((LANES,), jnp.float32)
    zeros_v = jnp.zeros((LANES,), jnp.float32)
    bufs = (buf0, buf1)
    sems = (sem0, sem1)

    def put(buf, chunk, vals):
        for j in range(CH // LANES):
            ids_c = ids_v[pl.ds(chunk * CH + j * LANES, LANES)]
            plsc.store_scatter(buf, [iota + j * LANES, ids_c], vals)

    def dma(buf, sem, chunk):
        return pltpu.make_async_copy(
            buf, out_hbm.at[pl.ds(base + chunk * CH, CH)], sem)

    for b in range(2):
        put(bufs[b], b, ones_v)
        dma(bufs[b], sems[b], b).start()

    @pl.loop(2, n_chunks, step=2)
    def _steady(c):
        for b in range(2):
            chunk = c + b
            dma(bufs[b], sems[b], chunk).wait()
            put(bufs[b], chunk - 2, zeros_v)
            put(bufs[b], chunk, ones_v)
            dma(bufs[b], sems[b], chunk).start()

    for b in range(2):
        dma(bufs[b], sems[b], 0).wait()


def kernel(input_ids) -> jnp.ndarray:
    B, L = input_ids.shape
    n = B * L
    ids_flat = input_ids.reshape(n).astype(jnp.int32)
    zeros_blk = jnp.zeros((CH, VOCAB), jnp.float32)
    rows_per_w = n // NW

    run = pl.kernel(
        _sc_onehot,
        out_type=jax.ShapeDtypeStruct((n, VOCAB), jnp.float32),
        mesh=plsc.VectorSubcoreMesh(
            core_axis_name="c", subcore_axis_name="s",
            num_cores=NC, num_subcores=NS),
        compiler_params=pltpu.CompilerParams(
            needs_layout_passes=False, use_tc_tiling_on_sc=True),
        scratch_types=[
            pltpu.VMEM((rows_per_w,), jnp.int32),
            pltpu.VMEM((CH, VOCAB), jnp.float32),
            pltpu.VMEM((CH, VOCAB), jnp.float32),
            pltpu.SemaphoreType.DMA,
            pltpu.SemaphoreType.DMA,
        ],
    )
    out = run(ids_flat, zeros_blk)
    return out.reshape(B, L, VOCAB)


# speedup vs baseline: 6.4354x; 6.4354x over previous
import jax
import jax.numpy as jnp
from jax.experimental import pallas as pl

VOCAB = 1000


def _onehot_block(ids_ref, out_ref):
    ids = ids_ref[0, 0, :]
    iota = jax.lax.broadcasted_iota(jnp.int32, (VOCAB, ids.shape[0]), 0)
    out_ref[0, :, :] = (iota == ids[None, :]).astype(jnp.float32)


def kernel(input_ids) -> jnp.ndarray:
    B, L = input_ids.shape
    ids_t = input_ids.T.reshape(L, 1, B).astype(jnp.int32)
    out = pl.pallas_call(
        _onehot_block,
        grid=(L,),
        in_specs=[pl.BlockSpec((1, 1, B), lambda i: (i, 0, 0))],
        out_specs=pl.BlockSpec((1, VOCAB, B), lambda i: (i, 0, 0)),
        out_shape=jax.ShapeDtypeStruct((L, VOCAB, B), jnp.float32),
    )(ids_t)
    return out.transpose(2, 0, 1)
